# baseline (speedup 1.0000x reference)
import jax
import jax.numpy as jnp
from jax.experimental import pallas as pl
from jax.experimental.pallas import tpu as pltpu

_C, _T, _H, _W = 3, 32, 224, 224
_S = _T // 4
_IDX = tuple((j * (_T - 1)) // (_S - 1) for j in range(_S))


_FPS = 8


def _gather_body(*refs):
    in_refs, out_ref = refs[:_FPS], refs[_FPS]
    for k in range(_FPS):
        out_ref[:, k] = in_refs[k][:, 0]


def _make_in_map(k):
    def in_map(j):
        return (0, ((j * _FPS + k) * (_T - 1)) // (_S - 1), 0, 0)

    return in_map


def kernel(frames):
    fast = jnp.copy(frames)
    slow = pl.pallas_call(
        _gather_body,
        grid=(_S // _FPS,),
        in_specs=[
            pl.BlockSpec((_C, 1, _H, _W), _make_in_map(k)) for k in range(_FPS)
        ],
        out_specs=pl.BlockSpec((_C, _FPS, _H, _W), lambda j: (0, j, 0, 0)),
        out_shape=jax.ShapeDtypeStruct((_C, _S, _H, _W), frames.dtype),
    )(frames, *([frames] * (_FPS - 1)))
    return slow, fast


# speedup vs baseline: 1.0390x; 1.0390x over previous
import jax
import jax.numpy as jnp
from jax.experimental import pallas as pl
from jax.experimental.pallas import tpu as pltpu

_C, _T, _H, _W = 3, 32, 224, 224
_S = _T // 4
_IDX = tuple((j * (_T - 1)) // (_S - 1) for j in range(_S))


_FPS = 4


def _gather_body(*refs):
    in_refs, out_ref = refs[:_FPS], refs[_FPS]
    for k in range(_FPS):
        out_ref[:, k] = in_refs[k][:, 0]


def _make_in_map(k):
    def in_map(j):
        return (0, ((j * _FPS + k) * (_T - 1)) // (_S - 1), 0, 0)

    return in_map


def kernel(frames):
    h = _T // 2
    fast = jnp.concatenate(
        [jnp.copy(frames[:, :h]), jnp.copy(frames[:, h:])], axis=1
    )
    slow = pl.pallas_call(
        _gather_body,
        grid=(_S // _FPS,),
        in_specs=[
            pl.BlockSpec((_C, 1, _H, _W), _make_in_map(k)) for k in range(_FPS)
        ],
        out_specs=pl.BlockSpec((_C, _FPS, _H, _W), lambda j: (0, j, 0, 0)),
        out_shape=jax.ShapeDtypeStruct((_C, _S, _H, _W), frames.dtype),
    )(frames, *([frames] * (_FPS - 1)))
    return slow, fast


# speedup vs baseline: 1.0403x; 1.0012x over previous
import jax
import jax.numpy as jnp
from jax.experimental import pallas as pl
from jax.experimental.pallas import tpu as pltpu

_C, _T, _H, _W = 3, 32, 224, 224
_S = _T // 4
_IDX = tuple((j * (_T - 1)) // (_S - 1) for j in range(_S))


_FPS = 4


def _gather_body(*refs):
    in_refs, out_ref = refs[:_FPS], refs[_FPS]
    for k in range(_FPS):
        out_ref[:, k] = in_refs[k][:, 0]


def _make_in_map(k):
    def in_map(j):
        return (0, ((j * _FPS + k) * (_T - 1)) // (_S - 1), 0, 0)

    return in_map


def kernel(frames):
    fast = jnp.copy(frames)
    slow = pl.pallas_call(
        _gather_body,
        grid=(_S // _FPS,),
        in_specs=[
            pl.BlockSpec((_C, 1, _H, _W), _make_in_map(k)) for k in range(_FPS)
        ],
        out_specs=pl.BlockSpec((_C, _FPS, _H, _W), lambda j: (0, j, 0, 0)),
        out_shape=jax.ShapeDtypeStruct((_C, _S, _H, _W), frames.dtype),
        cost_estimate=pl.CostEstimate(
            flops=0, transcendentals=0, bytes_accessed=2 * _C * _S * _H * _W * 4
        ),
    )(frames, *([frames] * (_FPS - 1)))
    return slow, fast
